# baseline (speedup 1.0000x reference)
import functools

import jax
import jax.numpy as jnp
from jax import lax
from jax.experimental import pallas as pl
from jax.experimental.pallas import tpu as pltpu
from jax.experimental.pallas import tpu_sc as plsc

NUM_ITEMS = 1000000
LATENT_DIM = 16
BATCH = 16384
HIST = 50

NC = 2
NS = 16
NW = NC * NS

TOTAL = BATCH * HIST
PER_W = TOTAL // NW
G = 128
CHUNK = 1024
GROUPS_PER_CHUNK = CHUNK // G
CHUNKS_PER_W = PER_W // CHUNK
GROUPS_PER_W = PER_W // G


def _sc_gather(idx2d, table_p, table_c):
  mesh = plsc.VectorSubcoreMesh(core_axis_name="c", subcore_axis_name="s")

  @functools.partial(
      pl.kernel,
      out_type=(
          jax.ShapeDtypeStruct((TOTAL, LATENT_DIM), jnp.float32),
          jax.ShapeDtypeStruct((TOTAL, LATENT_DIM), jnp.float32),
      ),
      mesh=mesh,
      compiler_params=pltpu.CompilerParams(use_tc_tiling_on_sc=False),
      scratch_types=[
          pltpu.VMEM((GROUPS_PER_CHUNK, G), jnp.int32),
          pltpu.VMEM((CHUNK, LATENT_DIM), jnp.float32),
          pltpu.VMEM((CHUNK, LATENT_DIM), jnp.float32),
          pltpu.SemaphoreType.DMA,
          pltpu.SemaphoreType.DMA,
      ],
  )
  def k(idx_hbm, tp_hbm, tc_hbm, outp_hbm, outc_hbm, idx_v, bufp, bufc,
        semp, semc):
    wid = lax.axis_index("s") * NC + lax.axis_index("c")
    wgbase = wid * GROUPS_PER_W
    wrbase = wid * PER_W

    def body(kk, carry):
      gbase = wgbase + kk * GROUPS_PER_CHUNK
      rbase = wrbase + kk * CHUNK
      pltpu.sync_copy(idx_hbm.at[pl.ds(gbase, GROUPS_PER_CHUNK)], idx_v)
      waits = []
      for j in range(GROUPS_PER_CHUNK):
        waits.append(
            pltpu.async_copy(tp_hbm.at[idx_v.at[j]],
                             bufp.at[pl.ds(j * G, G)], semp))
        waits.append(
            pltpu.async_copy(tc_hbm.at[idx_v.at[j]],
                             bufc.at[pl.ds(j * G, G)], semc))
      for w in waits:
        w.wait()
      pltpu.sync_copy(bufp, outp_hbm.at[pl.ds(rbase, CHUNK)])
      pltpu.sync_copy(bufc, outc_hbm.at[pl.ds(rbase, CHUNK)])
      return carry

    lax.fori_loop(0, CHUNKS_PER_W, body, 0)

  return k(idx2d, table_p, table_c)


def _rating_body(p_ref, c_ref, w_ref, b_ref, out_ref):
  s = p_ref[...] + c_ref[...]
  pred = jax.lax.dot_general(
      s, w_ref[...], (((1,), (0,)), ((), ())),
      preferred_element_type=jnp.float32)
  out_ref[...] = jax.nn.sigmoid(pred + b_ref[...])


def _tc_rating(gp, gc, affine_w, affine_b):
  blk = 8192
  grid = (TOTAL // blk,)
  return pl.pallas_call(
      _rating_body,
      grid=grid,
      in_specs=[
          pl.BlockSpec((blk, LATENT_DIM), lambda i: (i, 0)),
          pl.BlockSpec((blk, LATENT_DIM), lambda i: (i, 0)),
          pl.BlockSpec((LATENT_DIM, 1), lambda i: (0, 0)),
          pl.BlockSpec((1, 1), lambda i: (0, 0)),
      ],
      out_specs=pl.BlockSpec((blk, 1), lambda i: (i, 0)),
      out_shape=jax.ShapeDtypeStruct((TOTAL, 1), jnp.float32),
  )(gp, gc, affine_w, affine_b.reshape(1, 1))


def kernel(item_indices, item_personality_table, item_commonality_table,
           affine_W, affine_b):
  idx2d = item_indices.astype(jnp.int32).reshape(TOTAL // G, G)
  gp, gc = _sc_gather(idx2d, item_personality_table, item_commonality_table)
  rating = _tc_rating(gp, gc, affine_W, affine_b).reshape(BATCH, HIST, 1)
  return (
      rating,
      gp.reshape(BATCH, HIST, LATENT_DIM),
      gc.reshape(BATCH, HIST, LATENT_DIM),
  )


# speedup vs baseline: 1.2620x; 1.2620x over previous
import functools

import jax
import jax.numpy as jnp
from jax import lax
from jax.experimental import pallas as pl
from jax.experimental.pallas import tpu as pltpu
from jax.experimental.pallas import tpu_sc as plsc

NUM_ITEMS = 1000000
LATENT_DIM = 16
BATCH = 16384
HIST = 50

NC = 2
NS = 16
NW = NC * NS

TOTAL = BATCH * HIST
PER_W = TOTAL // NW
G = 128
CHUNK = 1024
GROUPS_PER_CHUNK = CHUNK // G
CHUNKS_PER_W = PER_W // CHUNK
GROUPS_PER_W = PER_W // G

BT = 4096
BB = 4096


def _t2(x):
  return jnp.swapaxes(x, 0, 1)


def _tin_body(tp_ref, tc_ref, op_ref, oc_ref):
  op_ref[...] = _t2(tp_ref[...])
  oc_ref[...] = _t2(tc_ref[...])


def _tc_transpose_in(table_p_t, table_c_t):
  grid = (pl.cdiv(NUM_ITEMS, BT),)
  return pl.pallas_call(
      _tin_body,
      grid=grid,
      in_specs=[
          pl.BlockSpec((LATENT_DIM, BT), lambda i: (0, i)),
          pl.BlockSpec((LATENT_DIM, BT), lambda i: (0, i)),
      ],
      out_specs=[
          pl.BlockSpec((BT, LATENT_DIM), lambda i: (i, 0)),
          pl.BlockSpec((BT, LATENT_DIM), lambda i: (i, 0)),
      ],
      out_shape=[
          jax.ShapeDtypeStruct((NUM_ITEMS, LATENT_DIM), jnp.float32),
          jax.ShapeDtypeStruct((NUM_ITEMS, LATENT_DIM), jnp.float32),
      ],
  )(table_p_t, table_c_t)


def _sc_gather(idx2d, table_p, table_c):
  mesh = plsc.VectorSubcoreMesh(core_axis_name="c", subcore_axis_name="s")

  @functools.partial(
      pl.kernel,
      out_type=(
          jax.ShapeDtypeStruct((TOTAL, LATENT_DIM), jnp.float32),
          jax.ShapeDtypeStruct((TOTAL, LATENT_DIM), jnp.float32),
      ),
      mesh=mesh,
      compiler_params=pltpu.CompilerParams(use_tc_tiling_on_sc=False),
      scratch_types=[
          pltpu.VMEM((GROUPS_PER_CHUNK, G), jnp.int32),
          pltpu.VMEM((CHUNK, LATENT_DIM), jnp.float32),
          pltpu.VMEM((CHUNK, LATENT_DIM), jnp.float32),
          pltpu.SemaphoreType.DMA,
          pltpu.SemaphoreType.DMA,
      ],
  )
  def k(idx_hbm, tp_hbm, tc_hbm, outp_hbm, outc_hbm, idx_v, bufp, bufc,
        semp, semc):
    wid = lax.axis_index("s") * NC + lax.axis_index("c")
    wgbase = wid * GROUPS_PER_W
    wrbase = wid * PER_W

    def body(kk, carry):
      gbase = wgbase + kk * GROUPS_PER_CHUNK
      rbase = wrbase + kk * CHUNK
      pltpu.sync_copy(idx_hbm.at[pl.ds(gbase, GROUPS_PER_CHUNK)], idx_v)
      waits = []
      for j in range(GROUPS_PER_CHUNK):
        waits.append(
            pltpu.async_copy(tp_hbm.at[idx_v.at[j]],
                             bufp.at[pl.ds(j * G, G)], semp))
        waits.append(
            pltpu.async_copy(tc_hbm.at[idx_v.at[j]],
                             bufc.at[pl.ds(j * G, G)], semc))
      for w in waits:
        w.wait()
      pltpu.sync_copy(bufp, outp_hbm.at[pl.ds(rbase, CHUNK)])
      pltpu.sync_copy(bufc, outc_hbm.at[pl.ds(rbase, CHUNK)])
      return carry

    lax.fori_loop(0, CHUNKS_PER_W, body, 0)

  return k(idx2d, table_p, table_c)


def _tout_body(gp_ref, gc_ref, w_ref, b_ref, op_ref, oc_ref, r_ref):
  pt = _t2(gp_ref[...])
  ct = _t2(gc_ref[...])
  op_ref[...] = pt
  oc_ref[...] = ct
  pred = jax.lax.dot_general(
      w_ref[...], pt + ct, (((1,), (0,)), ((), ())),
      preferred_element_type=jnp.float32)
  r_ref[...] = jax.nn.sigmoid(pred + b_ref[...]).reshape(1, 1, BB)


def _tc_transpose_out(gp_h, gc_h, w_t, b11):
  jb = BATCH // BB
  grid = (HIST, jb)
  return pl.pallas_call(
      _tout_body,
      grid=grid,
      in_specs=[
          pl.BlockSpec((BB, LATENT_DIM), lambda h, j: (h * jb + j, 0)),
          pl.BlockSpec((BB, LATENT_DIM), lambda h, j: (h * jb + j, 0)),
          pl.BlockSpec((1, LATENT_DIM), lambda h, j: (0, 0)),
          pl.BlockSpec((1, 1), lambda h, j: (0, 0)),
      ],
      out_specs=[
          pl.BlockSpec((LATENT_DIM, BB), lambda h, j: (h, j)),
          pl.BlockSpec((LATENT_DIM, BB), lambda h, j: (h, j)),
          pl.BlockSpec((1, 1, BB), lambda h, j: (h, 0, j)),
      ],
      out_shape=[
          jax.ShapeDtypeStruct((HIST * LATENT_DIM, BATCH), jnp.float32),
          jax.ShapeDtypeStruct((HIST * LATENT_DIM, BATCH), jnp.float32),
          jax.ShapeDtypeStruct((HIST, 1, BATCH), jnp.float32),
      ],
  )(gp_h, gc_h, w_t, b11)


def kernel(item_indices, item_personality_table, item_commonality_table,
           affine_W, affine_b):
  idx2d = item_indices.astype(jnp.int32).T.reshape(TOTAL // G, G)
  tp_lin, tc_lin = _tc_transpose_in(
      item_personality_table.T, item_commonality_table.T)
  gp_h, gc_h = _sc_gather(idx2d, tp_lin, tc_lin)
  gp_t, gc_t, rating_h = _tc_transpose_out(
      gp_h, gc_h, affine_W.T, affine_b.reshape(1, 1))
  rating = rating_h.transpose(2, 0, 1)
  gp = gp_t.reshape(HIST, LATENT_DIM, BATCH).transpose(2, 0, 1)
  gc = gc_t.reshape(HIST, LATENT_DIM, BATCH).transpose(2, 0, 1)
  return (rating, gp, gc)
